# baseline (speedup 1.0000x reference)
import functools

import jax
import jax.numpy as jnp
from jax import lax
from jax.experimental import pallas as pl
from jax.experimental.pallas import tpu as pltpu
from jax.experimental.pallas import tpu_sc as plsc


_K = 9
_TR = 256
_NW = 32
_RW = 128


def _tc_body(xrow_ref, xall_ref, wbd_ref, b_ref, out1_ref, y_ref, idx_ref):
    xrow = xrow_ref[0]
    xall = xall_ref[0]
    n = xall.shape[0]

    nall = jnp.sqrt(jnp.sum(xall * xall, axis=1, keepdims=True))
    xn_all = xall / jnp.maximum(nall, 1e-12)
    nrow = jnp.sqrt(jnp.sum(xrow * xrow, axis=1, keepdims=True))
    xn_row = xrow / jnp.maximum(nrow, 1e-12)
    inner = lax.dot_general(
        xn_row, xn_all, (((1,), (1,)), ((), ())),
        preferred_element_type=jnp.float32,
    )
    sq_all = jnp.sum(xn_all * xn_all, axis=1)
    sq_row = jnp.sum(xn_row * xn_row, axis=1)
    neg = -((sq_row[:, None] + (-2.0) * inner) + sq_all[None, :])

    zrow = lax.dot_general(
        xrow, wbd_ref[...], (((1,), (1,)), ((), ())),
        preferred_element_type=jnp.float32,
    )
    bvec = b_ref[0]
    out1_ref[0] = jnp.maximum(zrow[:, :48] + bvec[None, :48], 0.0)
    y_ref[0, :, :48] = zrow[:, 48:]

    iotaf = lax.broadcasted_iota(jnp.int32, (xrow.shape[0], n), 1).astype(jnp.float32)
    gbase = (pl.program_id(0) * n).astype(jnp.float32)
    for t in range(_K):
        m = jnp.max(neg, axis=1, keepdims=True)
        eqm = neg == m
        ids = jnp.where(eqm, iotaf, 0.0)
        idxf = jnp.max(ids, axis=1)
        neg = jnp.where(eqm, -3.0e38, neg)
        idx_ref[0, t, :] = (idxf + gbase).astype(jnp.int32)


def _sc_body(y_hbm, idx_hbm, b2_hbm, out2_hbm,
             idx_v, buf_v, cent_v, bias_v, out_v, sem0, sem1):
    cid = lax.axis_index("c")
    sid = lax.axis_index("s")
    wid = cid * 16 + sid
    base = wid * _RW

    pltpu.sync_copy(idx_hbm.at[wid], idx_v)
    pltpu.sync_copy(y_hbm.at[pl.ds(base, _RW)], cent_v)
    pltpu.sync_copy(b2_hbm, bias_v)

    def pre(i, carry):
        for c in range(3):
            sl = pl.ds(c * 16, 16)
            cent_v[i, sl] = cent_v[i, sl] - bias_v[sl]
        return carry

    lax.fori_loop(0, _RW, pre, 0)

    sems = [sem0, sem1]
    cp = pltpu.async_copy(y_hbm.at[idx_v.at[0]], buf_v.at[0], sems[0])
    for k in range(_K):
        cpn = None
        if k + 1 < _K:
            nb = (k + 1) % 2
            cpn = pltpu.async_copy(y_hbm.at[idx_v.at[k + 1]], buf_v.at[nb], sems[nb])
        cp.wait()
        kb = k % 2
        if k == 0:
            def body0(i, carry):
                for c in range(3):
                    sl = pl.ds(c * 16, 16)
                    out_v[i, sl] = jnp.maximum(buf_v[kb, i, sl] - cent_v[i, sl], 0.0)
                return carry
            lax.fori_loop(0, _RW, body0, 0)
        else:
            def bodyk(i, carry, kb=kb):
                for c in range(3):
                    sl = pl.ds(c * 16, 16)
                    out_v[i, sl] = out_v[i, sl] + jnp.maximum(
                        buf_v[kb, i, sl] - cent_v[i, sl], 0.0)
                return carry
            lax.fori_loop(0, _RW, bodyk, 0)
        cp = cpn

    def fin(i, carry):
        for c in range(3):
            sl = pl.ds(c * 16, 16)
            out_v[i, sl] = out_v[i, sl] * (1.0 / _K)
        return carry

    lax.fori_loop(0, _RW, fin, 0)
    pltpu.sync_copy(out_v, out2_hbm.at[pl.ds(base, _RW)])


def kernel(x, W, b):
    Bb, Cc, Hh, Ww = x.shape
    N = Hh * Ww
    Cout = W.shape[0]
    half = Cc // 2

    xt = jnp.transpose(x.reshape(Bb, Cc, N), (0, 2, 1))

    Wbd = jnp.zeros((Cout, Cc), dtype=W.dtype)
    Wbd = Wbd.at[0:24, 0:half].set(W[0:24])
    Wbd = Wbd.at[24:48, half:Cc].set(W[24:48])
    Wbd = Wbd.at[48:72, 0:half].set(W[48:72])
    Wbd = Wbd.at[72:96, half:Cc].set(W[72:96])

    nt = N // _TR
    out1, y, idx = pl.pallas_call(
        _tc_body,
        grid=(Bb, nt),
        in_specs=[
            pl.BlockSpec((1, _TR, Cc), lambda i, r: (i, r, 0)),
            pl.BlockSpec((1, N, Cc), lambda i, r: (i, 0, 0)),
            pl.BlockSpec((Cout, Cc), lambda i, r: (0, 0)),
            pl.BlockSpec((1, Cout), lambda i, r: (0, 0)),
        ],
        out_specs=[
            pl.BlockSpec((1, _TR, 48), lambda i, r: (i, r, 0)),
            pl.BlockSpec((1, _TR, 128), lambda i, r: (i, r, 0)),
            pl.BlockSpec((1, _K, _TR), lambda i, r: (i, 0, r)),
        ],
        out_shape=[
            jax.ShapeDtypeStruct((Bb, N, 48), jnp.float32),
            jax.ShapeDtypeStruct((Bb, N, 128), jnp.float32),
            jax.ShapeDtypeStruct((Bb, _K, N), jnp.int32),
        ],
    )(xt, xt, Wbd, b.reshape(1, Cout))

    chunks = N // _RW
    idxw = (
        idx.reshape(Bb, _K, chunks, _RW)
        .transpose(0, 2, 1, 3)
        .reshape(_NW, _K, _RW)
    )
    y2 = y.reshape(Bb * N, 128)

    mesh = plsc.VectorSubcoreMesh(
        core_axis_name="c", subcore_axis_name="s", num_cores=2, num_subcores=16
    )
    sc = functools.partial(
        pl.kernel,
        mesh=mesh,
        out_type=jax.ShapeDtypeStruct((Bb * N, 128), jnp.float32),
        scratch_types=[
            pltpu.VMEM((_K, _RW), jnp.int32),
            pltpu.VMEM((2, _RW, 128), jnp.float32),
            pltpu.VMEM((_RW, 128), jnp.float32),
            pltpu.VMEM((48,), jnp.float32),
            pltpu.VMEM((_RW, 128), jnp.float32),
            pltpu.SemaphoreType.DMA,
            pltpu.SemaphoreType.DMA,
        ],
    )(_sc_body)
    out2 = sc(y2, idxw, b[48:])

    out = jnp.concatenate([out1, out2.reshape(Bb, N, 128)[:, :, :48]], axis=2)
    return jnp.transpose(out, (0, 2, 1)).reshape(Bb, Cout, Hh, Ww)


# speedup vs baseline: 1.2029x; 1.2029x over previous
import jax
import jax.numpy as jnp
from jax import lax
from jax.experimental import pallas as pl


_K = 9
_TR = 256


def _body(xrow_ref, xall_ref, wbd_ref, b_ref, out_ref):
    xrow = xrow_ref[0]
    xall = xall_ref[0]

    nall = jnp.sqrt(jnp.sum(xall * xall, axis=1, keepdims=True))
    xn_all = xall / jnp.maximum(nall, 1e-12)
    nrow = jnp.sqrt(jnp.sum(xrow * xrow, axis=1, keepdims=True))
    xn_row = xrow / jnp.maximum(nrow, 1e-12)
    inner = lax.dot_general(
        xn_row, xn_all, (((1,), (1,)), ((), ())),
        preferred_element_type=jnp.float32,
    )
    sq_all = jnp.sum(xn_all * xn_all, axis=1)
    sq_row = jnp.sum(xn_row * xn_row, axis=1)
    neg = -((sq_row[:, None] + (-2.0) * inner) + sq_all[None, :])

    zrow = lax.dot_general(
        xrow, wbd_ref[...], (((1,), (1,)), ((), ())),
        preferred_element_type=jnp.float32,
    )
    y_all = lax.dot_general(
        xall, wbd_ref[pl.ds(48, 48), :], (((1,), (1,)), ((), ())),
        preferred_element_type=jnp.float32,
    )
    bvec = b_ref[0]
    out1 = jnp.maximum(zrow[:, :48] + bvec[None, :48], 0.0)
    y_row = zrow[:, 48:]
    b2 = bvec[None, 48:]

    y_hi = y_all.astype(jnp.bfloat16)
    y_lo = (y_all - y_hi.astype(jnp.float32)).astype(jnp.bfloat16)
    y_cat = jnp.concatenate([y_hi, y_lo], axis=1)

    acc = jnp.zeros_like(y_row)
    for _ in range(_K):
        m = jnp.max(neg, axis=1, keepdims=True)
        onehot = jnp.sign(neg - m) + 1.0
        neg = neg - onehot * 3.0e38
        yj2 = lax.dot_general(
            onehot.astype(jnp.bfloat16), y_cat, (((1,), (0,)), ((), ())),
            preferred_element_type=jnp.float32,
        )
        acc += jnp.maximum((yj2[:, :48] + yj2[:, 48:]) - y_row + b2, 0.0)

    out_ref[0] = jnp.concatenate([out1, acc * (1.0 / _K)], axis=1)


def kernel(x, W, b):
    Bb, Cc, Hh, Ww = x.shape
    N = Hh * Ww
    Cout = W.shape[0]
    half = Cc // 2

    xt = jnp.transpose(x.reshape(Bb, Cc, N), (0, 2, 1))

    Wbd = jnp.zeros((Cout, Cc), dtype=W.dtype)
    Wbd = Wbd.at[0:24, 0:half].set(W[0:24])
    Wbd = Wbd.at[24:48, half:Cc].set(W[24:48])
    Wbd = Wbd.at[48:72, 0:half].set(W[48:72])
    Wbd = Wbd.at[72:96, half:Cc].set(W[72:96])

    nt = N // _TR
    out = pl.pallas_call(
        _body,
        grid=(Bb, nt),
        in_specs=[
            pl.BlockSpec((1, _TR, Cc), lambda i, r: (i, r, 0)),
            pl.BlockSpec((1, N, Cc), lambda i, r: (i, 0, 0)),
            pl.BlockSpec((Cout, Cc), lambda i, r: (0, 0)),
            pl.BlockSpec((1, Cout), lambda i, r: (0, 0)),
        ],
        out_specs=pl.BlockSpec((1, _TR, Cout), lambda i, r: (i, r, 0)),
        out_shape=jax.ShapeDtypeStruct((Bb, N, Cout), jnp.float32),
    )(xt, xt, Wbd, b.reshape(1, Cout))

    return jnp.transpose(out, (0, 2, 1)).reshape(Bb, Cout, Hh, Ww)


# speedup vs baseline: 1.9984x; 1.6613x over previous
import jax
import jax.numpy as jnp
from jax import lax
from jax.experimental import pallas as pl


_K = 9
_TR = 512


def _body(xrow_ref, xall_ref, wbd_ref, b_ref, out_ref):
    xrow = xrow_ref[0]
    xall = xall_ref[0]

    nall = jnp.sqrt(jnp.sum(xall * xall, axis=1, keepdims=True))
    xn_all = xall / jnp.maximum(nall, 1e-12)
    nrow = jnp.sqrt(jnp.sum(xrow * xrow, axis=1, keepdims=True))
    xn_row = xrow / jnp.maximum(nrow, 1e-12)
    inner = lax.dot_general(
        xn_row, xn_all, (((1,), (1,)), ((), ())),
        preferred_element_type=jnp.float32,
    )
    sq_all = jnp.sum(xn_all * xn_all, axis=1)
    sq_row = jnp.sum(xn_row * xn_row, axis=1)
    neg = -((sq_row[:, None] + (-2.0) * inner) + sq_all[None, :])

    zrow = lax.dot_general(
        xrow, wbd_ref[...], (((1,), (1,)), ((), ())),
        preferred_element_type=jnp.float32,
    )
    y_all = lax.dot_general(
        xall, wbd_ref[pl.ds(48, 48), :], (((1,), (1,)), ((), ())),
        preferred_element_type=jnp.float32,
    )
    bvec = b_ref[0]
    out1 = jnp.maximum(zrow[:, :48] + bvec[None, :48], 0.0)
    y_row = zrow[:, 48:]
    b2 = bvec[None, 48:]

    acc = jnp.zeros_like(y_row)
    for _ in range(_K):
        m = jnp.max(neg, axis=1, keepdims=True)
        onehot = (neg == m).astype(jnp.float32)
        neg = neg - onehot * 3.0e38
        yj = lax.dot_general(
            onehot, y_all, (((1,), (0,)), ((), ())),
            preferred_element_type=jnp.float32,
        )
        acc += jnp.maximum(yj - y_row + b2, 0.0)

    out_ref[0] = jnp.concatenate([out1, acc * (1.0 / _K)], axis=1)


def kernel(x, W, b):
    Bb, Cc, Hh, Ww = x.shape
    N = Hh * Ww
    Cout = W.shape[0]
    half = Cc // 2

    xt = jnp.transpose(x.reshape(Bb, Cc, N), (0, 2, 1))

    Wbd = jnp.zeros((Cout, Cc), dtype=W.dtype)
    Wbd = Wbd.at[0:24, 0:half].set(W[0:24])
    Wbd = Wbd.at[24:48, half:Cc].set(W[24:48])
    Wbd = Wbd.at[48:72, 0:half].set(W[48:72])
    Wbd = Wbd.at[72:96, half:Cc].set(W[72:96])

    nt = N // _TR
    out = pl.pallas_call(
        _body,
        grid=(Bb, nt),
        in_specs=[
            pl.BlockSpec((1, _TR, Cc), lambda i, r: (i, r, 0)),
            pl.BlockSpec((1, N, Cc), lambda i, r: (i, 0, 0)),
            pl.BlockSpec((Cout, Cc), lambda i, r: (0, 0)),
            pl.BlockSpec((1, Cout), lambda i, r: (0, 0)),
        ],
        out_specs=pl.BlockSpec((1, _TR, Cout), lambda i, r: (i, r, 0)),
        out_shape=jax.ShapeDtypeStruct((Bb, N, Cout), jnp.float32),
    )(xt, xt, Wbd, b.reshape(1, Cout))

    return jnp.transpose(out, (0, 2, 1)).reshape(Bb, Cout, Hh, Ww)
